# baseline (speedup 1.0000x reference)
import functools

import jax
import jax.numpy as jnp
import numpy as np
from jax.experimental import pallas as pl
from jax.experimental.pallas import tpu as pltpu

_H = 16
_ST = 256


def _q_proj_kernel(qg_ref, wq_ref, bq_ref, out_ref):
    out_ref[0] = jax.lax.dot_general(
        qg_ref[0], wq_ref[...], (((1,), (1,)), ((), ())),
        preferred_element_type=jnp.float32) + bq_ref[...]


def _fused_kernel(u, up, s_len, n_kt, q_ref, k_ref, v_ref, wk_ref, wv_ref,
                  attn_ref, ctx_ref, kt_s, u_s):
    h_all = _H
    dh = wk_ref.shape[0] // h_all
    s = pl.program_id(1)

    @pl.when(s < n_kt)
    def _build_kt():
        tile = jax.lax.dot_general(
            wk_ref[...], k_ref[0], (((1,), (1,)), ((), ())),
            preferred_element_type=jnp.float32)
        kt_s[:, pl.ds(s * _ST, _ST)] = tile

    @pl.when(s == n_kt - 1)
    def _attention():
        rows1 = jax.lax.broadcasted_iota(jnp.int32, (up, 1), 0)
        rows = jax.lax.broadcasted_iota(jnp.int32, (up, up), 0)
        cols = jax.lax.broadcasted_iota(jnp.int32, (up, up), 1)
        for h in range(h_all):
            q = q_ref[0, h]
            kt = kt_s[h * dh:(h + 1) * dh, :]
            qs = q[:up]
            qf = q[up:]
            ss = jnp.dot(qs, kt, preferred_element_type=jnp.float32)
            m_col = (jnp.max(ss, axis=1, keepdims=True)
                     - jnp.sum(ss, axis=1, keepdims=True) / s_len)
            m_col = jnp.where(rows1 < u, m_col, jnp.float32(-1e30))
            m_row = jnp.transpose(m_col)
            beats = (m_row > m_col) | ((m_row == m_col) & (cols < rows))
            ranks = jnp.sum(beats.astype(jnp.int32), axis=1, keepdims=True)
            perm_mat = (rows == jnp.transpose(ranks)).astype(jnp.float32)

            fs = jnp.dot(qf, kt, preferred_element_type=jnp.float32)
            fs_max = jnp.max(fs, axis=1, keepdims=True)
            ex = jnp.exp(fs - fs_max)
            attn_f = ex / jnp.sum(ex, axis=1, keepdims=True)
            attn_p = jnp.dot(perm_mat, attn_f,
                             preferred_element_type=jnp.float32)
            attn_ref[0, h] = attn_p[:u]

    @pl.when(s >= n_kt)
    def _accum_u():
        tv = s - n_kt
        vblk = v_ref[0]
        for h in range(h_all):
            part = jnp.dot(attn_ref[0, h, :, pl.ds(tv * _ST, _ST)], vblk,
                           preferred_element_type=jnp.float32)

            @pl.when(tv == 0)
            def _init():
                u_s[h, :u] = part

            @pl.when(tv > 0)
            def _add():
                u_s[h, :u] += part

    @pl.when(s == 2 * n_kt - 1)
    def _context():
        for h in range(h_all):
            ctx_ref[0, h] = jax.lax.dot_general(
                u_s[h, :u], wv_ref[h * dh:(h + 1) * dh, :],
                (((1,), (1,)), ((), ())),
                preferred_element_type=jnp.float32)


def _out_proj_kernel(ctx_ref, wo_ref, bv_ref, bo_ref, out_ref):
    out_ref[0] = jax.lax.dot_general(
        ctx_ref[0] + bv_ref[...], wo_ref[...], (((1,), (1,)), ((), ())),
        preferred_element_type=jnp.float32) + bo_ref[...]


def kernel(queries, keys, values, Wq, bq, Wk, bk, Wv, bv, Wo, bo):
    del bk
    B, L, D = queries.shape
    S = keys.shape[1]
    H = _H
    dh = D // H
    u = min(5 * int(np.ceil(np.log(S))), L)
    up = (u + 7) // 8 * 8
    n_kt = S // _ST

    perm = jax.random.permutation(jax.random.key(42), L)[:u]
    pad = ((0, 0), (0, up - u), (0, 0))
    qg = jnp.concatenate([
        jnp.pad(queries[:, perm, :], pad),
        jnp.pad(queries[:, :u, :], pad),
    ], axis=1)

    q_proj = pl.pallas_call(
        _q_proj_kernel,
        grid=(B,),
        in_specs=[
            pl.BlockSpec((1, 2 * up, D), lambda b: (b, 0, 0)),
            pl.BlockSpec((D, D), lambda b: (0, 0)),
            pl.BlockSpec((1, D), lambda b: (0, 0)),
        ],
        out_specs=pl.BlockSpec((1, 2 * up, D), lambda b: (b, 0, 0)),
        out_shape=jax.ShapeDtypeStruct((B, 2 * up, D), jnp.float32),
        compiler_params=pltpu.CompilerParams(
            dimension_semantics=("parallel",)),
    )(qg, Wq, bq.reshape(1, D))
    q_heads = q_proj.reshape(B, 2 * up, H, dh).transpose(0, 2, 1, 3)

    attn, ctx = pl.pallas_call(
        functools.partial(_fused_kernel, u, up, S, n_kt),
        grid=(B, 2 * n_kt),
        in_specs=[
            pl.BlockSpec((1, H, 2 * up, dh), lambda b, s: (b, 0, 0, 0)),
            pl.BlockSpec((1, _ST, D),
                         lambda b, s: (b, jnp.minimum(s, n_kt - 1), 0)),
            pl.BlockSpec((1, _ST, D),
                         lambda b, s: (b, jnp.maximum(s - n_kt, 0), 0)),
            pl.BlockSpec((D, D), lambda b, s: (0, 0)),
            pl.BlockSpec((D, D), lambda b, s: (0, 0)),
        ],
        out_specs=[
            pl.BlockSpec((1, H, u, S), lambda b, s: (b, 0, 0, 0)),
            pl.BlockSpec((1, H, u, dh), lambda b, s: (b, 0, 0, 0)),
        ],
        out_shape=[
            jax.ShapeDtypeStruct((B, H, u, S), jnp.float32),
            jax.ShapeDtypeStruct((B, H, u, dh), jnp.float32),
        ],
        scratch_shapes=[
            pltpu.VMEM((D, S), jnp.float32),
            pltpu.VMEM((H, up, D), jnp.float32),
        ],
        compiler_params=pltpu.CompilerParams(
            dimension_semantics=("parallel", "arbitrary")),
    )(q_heads, keys, values, Wk, Wv)

    ctx_all = ctx.transpose(0, 2, 1, 3).reshape(B, u, D)
    out = pl.pallas_call(
        _out_proj_kernel,
        grid=(B,),
        in_specs=[
            pl.BlockSpec((1, u, D), lambda b: (b, 0, 0)),
            pl.BlockSpec((D, D), lambda b: (0, 0)),
            pl.BlockSpec((1, D), lambda b: (0, 0)),
            pl.BlockSpec((1, D), lambda b: (0, 0)),
        ],
        out_specs=pl.BlockSpec((1, u, D), lambda b: (b, 0, 0)),
        out_shape=jax.ShapeDtypeStruct((B, u, D), jnp.float32),
        compiler_params=pltpu.CompilerParams(
            dimension_semantics=("parallel",)),
    )(ctx_all, Wo, bv.reshape(1, D), bo.reshape(1, D))

    return (out, attn)


# speedup vs baseline: 1.3796x; 1.3796x over previous
import functools

import jax
import jax.numpy as jnp
import numpy as np
from jax.experimental import pallas as pl
from jax.experimental.pallas import tpu as pltpu

_H = 16
_ST_K = 512
_ST_V = 2048


def _q_proj_kernel(qg_ref, wq_ref, bq_ref, out_ref):
    out_ref[0] = jax.lax.dot_general(
        qg_ref[0], wq_ref[...], (((1,), (1,)), ((), ())),
        preferred_element_type=jnp.float32) + bq_ref[...]


def _score_kernel(u, up, s_len, n_kt, q_ref, k_ref, wk_ref, attn_ref, kt_s):
    h_all = _H
    dh = wk_ref.shape[0] // h_all
    s = pl.program_id(1)

    tile = jax.lax.dot_general(
        wk_ref[...], k_ref[0], (((1,), (1,)), ((), ())),
        preferred_element_type=jnp.float32)
    kt_s[:, pl.ds(s * _ST_K, _ST_K)] = tile

    @pl.when(s == n_kt - 1)
    def _attention():
        rows1 = jax.lax.broadcasted_iota(jnp.int32, (up, 1), 0)
        rows = jax.lax.broadcasted_iota(jnp.int32, (up, up), 0)
        cols = jax.lax.broadcasted_iota(jnp.int32, (up, up), 1)
        for h in range(h_all):
            q = q_ref[0, h]
            kt = kt_s[h * dh:(h + 1) * dh, :]
            qs = q[:up]
            qf = q[up:]
            ss = jnp.dot(qs, kt, preferred_element_type=jnp.float32)
            m_col = (jnp.max(ss, axis=1, keepdims=True)
                     - jnp.sum(ss, axis=1, keepdims=True) / s_len)
            m_col = jnp.where(rows1 < u, m_col, jnp.float32(-1e30))
            m_row = jnp.transpose(m_col)
            beats = (m_row > m_col) | ((m_row == m_col) & (cols < rows))
            ranks = jnp.sum(beats.astype(jnp.int32), axis=1, keepdims=True)
            perm_mat = (rows == jnp.transpose(ranks)).astype(jnp.float32)

            fs = jnp.dot(qf, kt, preferred_element_type=jnp.float32)
            fs_max = jnp.max(fs, axis=1, keepdims=True)
            ex = jnp.exp(fs - fs_max)
            attn_f = ex / jnp.sum(ex, axis=1, keepdims=True)
            attn_p = jnp.dot(perm_mat, attn_f,
                             preferred_element_type=jnp.float32)
            attn_ref[0, h] = attn_p[:u]


def _ctx_kernel(u, n_vt, attn_ref, v_ref, wv_ref, wo_ref, bv_ref, bo_ref,
                out_ref, u_s):
    h_all = _H
    dh = wv_ref.shape[0] // h_all
    t = pl.program_id(1)
    vblk = v_ref[0]

    for h in range(h_all):
        part = jnp.dot(attn_ref[0, h, :, pl.ds(t * _ST_V, _ST_V)], vblk,
                       preferred_element_type=jnp.float32)

        @pl.when(t == 0)
        def _init():
            u_s[h, :u] = part

        @pl.when(t > 0)
        def _add():
            u_s[h, :u] += part

    @pl.when(t == n_vt - 1)
    def _finish():
        acc = jax.lax.dot_general(
            bv_ref[...], wo_ref[...], (((1,), (1,)), ((), ())),
            preferred_element_type=jnp.float32) + bo_ref[...]
        acc = jnp.broadcast_to(acc, (u, wo_ref.shape[0]))
        for h in range(h_all):
            ctx_h = jax.lax.dot_general(
                u_s[h, :u], wv_ref[h * dh:(h + 1) * dh, :],
                (((1,), (1,)), ((), ())),
                preferred_element_type=jnp.float32)
            acc = acc + jnp.dot(
                ctx_h, jnp.transpose(wo_ref[:, h * dh:(h + 1) * dh]),
                preferred_element_type=jnp.float32)
        out_ref[0] = acc


def kernel(queries, keys, values, Wq, bq, Wk, bk, Wv, bv, Wo, bo):
    del bk
    B, L, D = queries.shape
    S = keys.shape[1]
    H = _H
    dh = D // H
    u = min(5 * int(np.ceil(np.log(S))), L)
    up = (u + 7) // 8 * 8
    n_kt = S // _ST_K
    n_vt = S // _ST_V

    perm = jax.random.permutation(jax.random.key(42), L)[:u]
    pad = ((0, 0), (0, up - u), (0, 0))
    qg = jnp.concatenate([
        jnp.pad(queries[:, perm, :], pad),
        jnp.pad(queries[:, :u, :], pad),
    ], axis=1)

    q_proj = pl.pallas_call(
        _q_proj_kernel,
        grid=(B,),
        in_specs=[
            pl.BlockSpec((1, 2 * up, D), lambda b: (b, 0, 0)),
            pl.BlockSpec((D, D), lambda b: (0, 0)),
            pl.BlockSpec((1, D), lambda b: (0, 0)),
        ],
        out_specs=pl.BlockSpec((1, 2 * up, D), lambda b: (b, 0, 0)),
        out_shape=jax.ShapeDtypeStruct((B, 2 * up, D), jnp.float32),
        compiler_params=pltpu.CompilerParams(
            dimension_semantics=("parallel",)),
    )(qg, Wq, bq.reshape(1, D))
    q_heads = q_proj.reshape(B, 2 * up, H, dh).transpose(0, 2, 1, 3)

    attn = pl.pallas_call(
        functools.partial(_score_kernel, u, up, S, n_kt),
        grid=(B, n_kt),
        in_specs=[
            pl.BlockSpec((1, H, 2 * up, dh), lambda b, s: (b, 0, 0, 0)),
            pl.BlockSpec((1, _ST_K, D), lambda b, s: (b, s, 0)),
            pl.BlockSpec((D, D), lambda b, s: (0, 0)),
        ],
        out_specs=pl.BlockSpec((1, H, u, S), lambda b, s: (b, 0, 0, 0)),
        out_shape=jax.ShapeDtypeStruct((B, H, u, S), jnp.float32),
        scratch_shapes=[pltpu.VMEM((D, S), jnp.float32)],
        compiler_params=pltpu.CompilerParams(
            dimension_semantics=("parallel", "arbitrary")),
    )(q_heads, keys, Wk)

    out = pl.pallas_call(
        functools.partial(_ctx_kernel, u, n_vt),
        grid=(B, n_vt),
        in_specs=[
            pl.BlockSpec((1, H, u, S), lambda b, t: (b, 0, 0, 0)),
            pl.BlockSpec((1, _ST_V, D), lambda b, t: (b, t, 0)),
            pl.BlockSpec((D, D), lambda b, t: (0, 0)),
            pl.BlockSpec((D, D), lambda b, t: (0, 0)),
            pl.BlockSpec((1, D), lambda b, t: (0, 0)),
            pl.BlockSpec((1, D), lambda b, t: (0, 0)),
        ],
        out_specs=pl.BlockSpec((1, u, D), lambda b, t: (b, 0, 0)),
        out_shape=jax.ShapeDtypeStruct((B, u, D), jnp.float32),
        scratch_shapes=[pltpu.VMEM((H, up, D), jnp.float32)],
        compiler_params=pltpu.CompilerParams(
            dimension_semantics=("parallel", "arbitrary")),
    )(attn, values, Wv, Wo, bv.reshape(1, D), bo.reshape(1, D))

    return (out, attn)
